# baseline (speedup 1.0000x reference)
import jax
import jax.numpy as jnp
from jax import lax
from jax.experimental import pallas as pl
from jax.experimental.pallas import tpu as pltpu
from jax.experimental.pallas import tpu_sc as plsc

NC = 2
NS = 16
C = 128


def _fill(ref, rows, d, val):
    v16 = jnp.full((16,), val, jnp.float32)

    def fr(r, _):
        def fc(i, _):
            ref[r, pl.ds(i * 16, 16)] = v16
            return 0
        lax.fori_loop(0, d // 16, fc, 0)
        return 0
    lax.fori_loop(0, rows, fr, 0)


def _make_sc_agg(n, iters, d):
    rows_per_sub = n // NS
    nz = rows_per_sub // C

    mesh = plsc.VectorSubcoreMesh(core_axis_name="c", subcore_axis_name="s",
                                  num_cores=NC, num_subcores=NS)

    def body(x_hbm, eidx_hbm, agg_out, ib0, ib1, rows_0, rows_1, acc,
             isem_0, isem_1, gsem_0, gsem_1):
        cid = lax.axis_index("c")
        sid = lax.axis_index("s")
        wid = cid * NS + sid

        ib = (ib0, ib1)
        rows = (rows_0, rows_1)
        isem = (isem_0, isem_1)
        gsem = (gsem_0, gsem_1)

        _fill(rows_0, C, d, 0.0)
        row0 = sid * rows_per_sub

        def zero_acc(k, _):
            pltpu.sync_copy(rows_0, acc.at[pl.ds(row0 + k * C, C)])
            return 0
        lax.fori_loop(0, nz, zero_acc, 0)
        plsc.subcore_barrier()

        dummy_r = x_hbm.at[pl.ds(0, C)]
        dummy_i = eidx_hbm.at[wid, 0]

        pltpu.sync_copy(eidx_hbm.at[wid, 0], ib[0])
        pltpu.async_copy(eidx_hbm.at[wid, 1], ib[1], isem[1])
        pltpu.async_copy(x_hbm.at[ib[0].at[0]], rows_0, gsem[0])

        def phase(j, b, fire_gather, fire_idx):
            o = 1 - b
            if fire_gather:
                pltpu.make_async_copy(dummy_i, ib[o], isem[o]).wait()
                pltpu.async_copy(x_hbm.at[ib[o].at[0]], rows[o], gsem[o])
            pltpu.make_async_copy(dummy_r, rows[b], gsem[b]).wait()
            pltpu.sync_copy(rows[b], acc.at[ib[b].at[1]], add=True)
            if fire_idx:
                pltpu.async_copy(eidx_hbm.at[wid, j + 2], ib[b], isem[b])

        def step(k, _):
            phase(2 * k, 0, True, True)
            phase(2 * k + 1, 1, True, True)
            return 0
        lax.fori_loop(0, (iters - 2) // 2, step, 0)

        phase(iters - 2, (iters - 2) % 2, True, False)
        phase(iters - 1, (iters - 1) % 2, False, False)
        plsc.subcore_barrier()

        pltpu.sync_copy(acc.at[pl.ds(row0, rows_per_sub)],
                        agg_out.at[cid, pl.ds(row0, rows_per_sub)])

    return pl.kernel(
        body,
        out_type=jax.ShapeDtypeStruct((NC, n, d), jnp.float32),
        mesh=mesh,
        scratch_types=[
            pltpu.VMEM((2, C), jnp.int32),
            pltpu.VMEM((2, C), jnp.int32),
            pltpu.VMEM((C, d), jnp.float32),
            pltpu.VMEM((C, d), jnp.float32),
            pltpu.VMEM_SHARED((n, d), jnp.float32),
            pltpu.SemaphoreType.DMA,
            pltpu.SemaphoreType.DMA,
            pltpu.SemaphoreType.DMA,
            pltpu.SemaphoreType.DMA,
        ],
    )


def _make_sc_counts(n, iters, d):
    rows_per_sub = n // NS
    nz = rows_per_sub // C
    burst = 5
    nb = iters // burst

    mesh = plsc.VectorSubcoreMesh(core_axis_name="c", subcore_axis_name="s",
                                  num_cores=NC, num_subcores=NS)

    def body(dst_hbm, cnt_out, didx_v, ones_v, cacc, sem_s):
        cid = lax.axis_index("c")
        sid = lax.axis_index("s")
        wid = cid * NS + sid

        pltpu.sync_copy(dst_hbm.at[wid], didx_v)

        _fill(ones_v, C, d, 0.0)
        row0 = sid * rows_per_sub

        def zero_acc(k, _):
            pltpu.sync_copy(ones_v, cacc.at[pl.ds(row0 + k * C, C)])
            return 0
        lax.fori_loop(0, nz, zero_acc, 0)
        _fill(ones_v, C, d, 1.0)
        plsc.subcore_barrier()

        def step(k, _):
            descs = [
                pltpu.async_copy(ones_v, cacc.at[didx_v.at[burst * k + t]],
                                 sem_s, add=True)
                for t in range(burst)
            ]
            for dsc in descs:
                dsc.wait()
            return 0
        lax.fori_loop(0, nb, step, 0)
        plsc.subcore_barrier()

        pltpu.sync_copy(cacc.at[pl.ds(row0, rows_per_sub)],
                        cnt_out.at[cid, pl.ds(row0, rows_per_sub)])

    return pl.kernel(
        body,
        out_type=jax.ShapeDtypeStruct((NC, n, d), jnp.float32),
        mesh=mesh,
        scratch_types=[
            pltpu.VMEM((iters, C), jnp.int32),
            pltpu.VMEM((C, d), jnp.float32),
            pltpu.VMEM_SHARED((n, d), jnp.float32),
            pltpu.SemaphoreType.DMA,
        ],
    )


def _dense(aggp, cntp, xin, wl, wr, b):
    n, d = xin.shape
    h = wl.shape[1]
    blk = min(1024, n)
    grid = (n // blk,)

    def body(aggp_ref, cnt_ref, x_ref, wl_ref, wr_ref, b_ref, o_ref):
        agg = aggp_ref[0] + aggp_ref[1]
        cnt = cnt_ref[0] + cnt_ref[1]
        mean = agg / jnp.maximum(cnt[:, :1], 1.0)
        acc = jnp.dot(mean, wl_ref[...], preferred_element_type=jnp.float32)
        acc = acc + jnp.dot(x_ref[...], wr_ref[...],
                            preferred_element_type=jnp.float32)
        acc = acc + b_ref[...]
        o_ref[...] = jnp.maximum(acc, 0.0)

    return pl.pallas_call(
        body,
        grid=grid,
        in_specs=[
            pl.BlockSpec((NC, blk, h), lambda i: (0, i, 0)),
            pl.BlockSpec((NC, blk, h), lambda i: (0, i, 0)),
            pl.BlockSpec((blk, d), lambda i: (i, 0)),
            pl.BlockSpec((d, h), lambda i: (0, 0)),
            pl.BlockSpec((d, h), lambda i: (0, 0)),
            pl.BlockSpec((1, h), lambda i: (0, 0)),
        ],
        out_specs=pl.BlockSpec((blk, h), lambda i: (i, 0)),
        out_shape=jax.ShapeDtypeStruct((n, h), jnp.float32),
    )(aggp, cntp, xin, wl, wr, b.reshape(1, h))


def kernel(x, edge_index, Wl1, Wr1, b1, Wl2, Wr2, b2):
    n, d = x.shape
    e = edge_index.shape[1]
    nw = NC * NS
    ew = e // nw
    iters = (ew + C - 1) // C
    iters = iters + (iters % 2)
    ewp = iters * C

    npad = ((n + NS * 8 - 1) // (NS * 8)) * (NS * 8)
    npad = max(npad, ((n + 1023) // 1024) * 1024)
    x_p = jnp.pad(x, ((0, npad - n), (0, 0)))

    srcw = jnp.pad(edge_index[0].reshape(nw, ew), ((0, 0), (0, ewp - ew)))
    dstw = jnp.pad(edge_index[1].reshape(nw, ew), ((0, 0), (0, ewp - ew)),
                   constant_values=n)
    eidx = jnp.stack([srcw.reshape(nw, iters, C),
                      dstw.reshape(nw, iters, C)], axis=2)
    dst3 = dstw.reshape(nw, iters, C)

    cnt = _make_sc_counts(npad, iters, d)(dst3)
    agg1 = _make_sc_agg(npad, iters, d)(x_p, eidx)
    h1 = _dense(agg1, cnt, x_p, Wl1, Wr1, b1)
    agg2 = _make_sc_agg(npad, iters, Wl1.shape[1])(h1, eidx)
    out = _dense(agg2, cnt, h1, Wl2, Wr2, b2)
    return out[:n]


# speedup vs baseline: 1.0407x; 1.0407x over previous
import jax
import jax.numpy as jnp
from jax import lax
from jax.experimental import pallas as pl
from jax.experimental.pallas import tpu as pltpu
from jax.experimental.pallas import tpu_sc as plsc

NC = 2
NS = 16
C = 128


def _fill(ref, rows, d, val):
    v16 = jnp.full((16,), val, jnp.float32)

    def fr(r, _):
        def fc(i, _):
            ref[r, pl.ds(i * 16, 16)] = v16
            return 0
        lax.fori_loop(0, d // 16, fc, 0)
        return 0
    lax.fori_loop(0, rows, fr, 0)


def _make_sc_agg(n, iters, d):
    rows_per_sub = n // NS
    nz = rows_per_sub // C

    mesh = plsc.VectorSubcoreMesh(core_axis_name="c", subcore_axis_name="s",
                                  num_cores=NC, num_subcores=NS)

    def body(x_hbm, src_hbm, dst_hbm, agg_out, sb0, sb1, sb2, sb3, didx_v,
             rows_0, rows_1, acc, ss0, ss1, ss2, ss3, gsem_0, gsem_1):
        cid = lax.axis_index("c")
        sid = lax.axis_index("s")
        wid = cid * NS + sid

        sb = (sb0, sb1, sb2, sb3)
        ssem = (ss0, ss1, ss2, ss3)
        rows = (rows_0, rows_1)
        gsem = (gsem_0, gsem_1)

        pltpu.sync_copy(dst_hbm.at[wid], didx_v)

        _fill(rows_0, C, d, 0.0)
        row0 = sid * rows_per_sub

        def zero_acc(k, _):
            pltpu.sync_copy(rows_0, acc.at[pl.ds(row0 + k * C, C)])
            return 0
        lax.fori_loop(0, nz, zero_acc, 0)
        plsc.subcore_barrier()

        dummy_r = x_hbm.at[pl.ds(0, C)]
        dummy_i = src_hbm.at[0, pl.ds(0, C)]

        pltpu.sync_copy(src_hbm.at[wid, pl.ds(0, C)], sb[0])
        for t in (1, 2, 3):
            pltpu.async_copy(src_hbm.at[wid, pl.ds(t * C, C)], sb[t],
                             ssem[t])
        pltpu.async_copy(x_hbm.at[sb[0]], rows_0, gsem[0])

        def phase(j, rb, si, fire_gather, fire_idx):
            if fire_gather:
                pltpu.make_async_copy(dummy_i, sb[(si + 1) % 4],
                                      ssem[(si + 1) % 4]).wait()
                pltpu.async_copy(x_hbm.at[sb[(si + 1) % 4]], rows[1 - rb],
                                 gsem[1 - rb])
            pltpu.make_async_copy(dummy_r, rows[rb], gsem[rb]).wait()
            if fire_idx:
                pltpu.async_copy(src_hbm.at[wid, pl.ds((j + 4) * C, C)],
                                 sb[si], ssem[si])
            pltpu.sync_copy(rows[rb], acc.at[didx_v.at[j]], add=True)

        def step(k, _):
            j = 4 * k
            phase(j, 0, 0, True, True)
            phase(j + 1, 1, 1, True, True)
            phase(j + 2, 0, 2, True, True)
            phase(j + 3, 1, 3, True, True)
            return 0
        lax.fori_loop(0, (iters - 4) // 4, step, 0)

        for j in range(iters - 4, iters):
            phase(j, j % 2, j % 4, j + 1 < iters, False)
        plsc.subcore_barrier()

        pltpu.sync_copy(acc.at[pl.ds(row0, rows_per_sub)],
                        agg_out.at[cid, pl.ds(row0, rows_per_sub)])

    return pl.kernel(
        body,
        out_type=jax.ShapeDtypeStruct((NC, n, d), jnp.float32),
        mesh=mesh,
        scratch_types=[
            pltpu.VMEM((C,), jnp.int32),
            pltpu.VMEM((C,), jnp.int32),
            pltpu.VMEM((C,), jnp.int32),
            pltpu.VMEM((C,), jnp.int32),
            pltpu.VMEM((iters, C), jnp.int32),
            pltpu.VMEM((C, d), jnp.float32),
            pltpu.VMEM((C, d), jnp.float32),
            pltpu.VMEM_SHARED((n, d), jnp.float32),
            pltpu.SemaphoreType.DMA,
            pltpu.SemaphoreType.DMA,
            pltpu.SemaphoreType.DMA,
            pltpu.SemaphoreType.DMA,
            pltpu.SemaphoreType.DMA,
            pltpu.SemaphoreType.DMA,
        ],
    )


def _make_sc_counts(n, iters, d):
    rows_per_sub = n // NS
    nz = rows_per_sub // C
    burst = 5
    nb = iters // burst

    mesh = plsc.VectorSubcoreMesh(core_axis_name="c", subcore_axis_name="s",
                                  num_cores=NC, num_subcores=NS)

    def body(dst_hbm, cnt_out, didx_v, ones_v, cacc, sem_s):
        cid = lax.axis_index("c")
        sid = lax.axis_index("s")
        wid = cid * NS + sid

        pltpu.sync_copy(dst_hbm.at[wid], didx_v)

        _fill(ones_v, C, d, 0.0)
        row0 = sid * rows_per_sub

        def zero_acc(k, _):
            pltpu.sync_copy(ones_v, cacc.at[pl.ds(row0 + k * C, C)])
            return 0
        lax.fori_loop(0, nz, zero_acc, 0)
        _fill(ones_v, C, d, 1.0)
        plsc.subcore_barrier()

        def step(k, _):
            descs = [
                pltpu.async_copy(ones_v, cacc.at[didx_v.at[burst * k + t]],
                                 sem_s, add=True)
                for t in range(burst)
            ]
            for dsc in descs:
                dsc.wait()
            return 0
        lax.fori_loop(0, nb, step, 0)
        plsc.subcore_barrier()

        pltpu.sync_copy(cacc.at[pl.ds(row0, rows_per_sub)],
                        cnt_out.at[cid, pl.ds(row0, rows_per_sub)])

    return pl.kernel(
        body,
        out_type=jax.ShapeDtypeStruct((NC, n, d), jnp.float32),
        mesh=mesh,
        scratch_types=[
            pltpu.VMEM((iters, C), jnp.int32),
            pltpu.VMEM((C, d), jnp.float32),
            pltpu.VMEM_SHARED((n, d), jnp.float32),
            pltpu.SemaphoreType.DMA,
        ],
    )


def _dense(aggp, cntp, xin, wl, wr, b):
    n, d = xin.shape
    h = wl.shape[1]
    blk = min(1024, n)
    grid = (n // blk,)

    def body(aggp_ref, cnt_ref, x_ref, wl_ref, wr_ref, b_ref, o_ref):
        agg = aggp_ref[0] + aggp_ref[1]
        cnt = cnt_ref[0] + cnt_ref[1]
        mean = agg / jnp.maximum(cnt[:, :1], 1.0)
        acc = jnp.dot(mean, wl_ref[...], preferred_element_type=jnp.float32)
        acc = acc + jnp.dot(x_ref[...], wr_ref[...],
                            preferred_element_type=jnp.float32)
        acc = acc + b_ref[...]
        o_ref[...] = jnp.maximum(acc, 0.0)

    return pl.pallas_call(
        body,
        grid=grid,
        in_specs=[
            pl.BlockSpec((NC, blk, h), lambda i: (0, i, 0)),
            pl.BlockSpec((NC, blk, h), lambda i: (0, i, 0)),
            pl.BlockSpec((blk, d), lambda i: (i, 0)),
            pl.BlockSpec((d, h), lambda i: (0, 0)),
            pl.BlockSpec((d, h), lambda i: (0, 0)),
            pl.BlockSpec((1, h), lambda i: (0, 0)),
        ],
        out_specs=pl.BlockSpec((blk, h), lambda i: (i, 0)),
        out_shape=jax.ShapeDtypeStruct((n, h), jnp.float32),
    )(aggp, cntp, xin, wl, wr, b.reshape(1, h))


def kernel(x, edge_index, Wl1, Wr1, b1, Wl2, Wr2, b2):
    n, d = x.shape
    e = edge_index.shape[1]
    nw = NC * NS
    ew = e // nw
    iters = (ew + C - 1) // C
    iters = ((iters + 3) // 4) * 4
    ewp = iters * C

    npad = ((n + NS * 8 - 1) // (NS * 8)) * (NS * 8)
    npad = max(npad, ((n + 1023) // 1024) * 1024)
    x_p = jnp.pad(x, ((0, npad - n), (0, 0)))

    srcw = jnp.pad(edge_index[0].reshape(nw, ew), ((0, 0), (0, ewp - ew)))
    dstw = jnp.pad(edge_index[1].reshape(nw, ew), ((0, 0), (0, ewp - ew)),
                   constant_values=n)
    dst3 = dstw.reshape(nw, iters, C)

    cnt = _make_sc_counts(npad, iters, d)(dst3)
    agg1 = _make_sc_agg(npad, iters, d)(x_p, srcw, dst3)
    h1 = _dense(agg1, cnt, x_p, Wl1, Wr1, b1)
    agg2 = _make_sc_agg(npad, iters, Wl1.shape[1])(h1, srcw, dst3)
    out = _dense(agg2, cnt, h1, Wl2, Wr2, b2)
    return out[:n]


# speedup vs baseline: 2.5940x; 2.4926x over previous
import jax
import jax.numpy as jnp
from jax import lax
from jax.experimental import pallas as pl
from jax.experimental.pallas import tpu as pltpu
from jax.experimental.pallas import tpu_sc as plsc

NC = 2
NS = 16
C = 80


def _fill(ref, rows, d, val):
    v16 = jnp.full((16,), val, jnp.float32)

    def fr(r, _):
        def fc(i, _):
            ref[r, pl.ds(i * 16, 16)] = v16
            return 0
        lax.fori_loop(0, d // 16, fc, 0)
        return 0
    lax.fori_loop(0, rows, fr, 0)


def _make_sc_agg(n, iters, d):
    rows_per_sub = n // NS
    nz = rows_per_sub // C

    mesh = plsc.VectorSubcoreMesh(core_axis_name="c", subcore_axis_name="s",
                                  num_cores=NC, num_subcores=NS)

    def body(x_hbm, src_hbm, dst_hbm, agg_out, sb0, sb1, sb2, sb3, didx_v,
             rows_0, rows_1, acc, ss0, ss1, ss2, ss3, gsem_0, gsem_1):
        cid = lax.axis_index("c")
        sid = lax.axis_index("s")
        wid = cid * NS + sid

        sb = (sb0, sb1, sb2, sb3)
        ssem = (ss0, ss1, ss2, ss3)
        rows = (rows_0, rows_1)
        gsem = (gsem_0, gsem_1)

        pltpu.sync_copy(dst_hbm.at[wid], didx_v)

        _fill(rows_0, C, d, 0.0)
        row0 = sid * rows_per_sub

        def zero_acc(k, _):
            pltpu.sync_copy(rows_0, acc.at[pl.ds(row0 + k * C, C)])
            return 0
        lax.fori_loop(0, nz, zero_acc, 0)
        plsc.subcore_barrier()

        dummy_r = x_hbm.at[pl.ds(0, C)]
        dummy_i = src_hbm.at[pl.ds(0, C)]
        ebase = wid * iters * C

        pltpu.sync_copy(src_hbm.at[pl.ds(ebase, C)], sb[0])
        for t in (1, 2, 3):
            pltpu.async_copy(src_hbm.at[pl.ds(ebase + t * C, C)], sb[t],
                             ssem[t])
        pltpu.async_copy(x_hbm.at[sb[0]], rows_0, gsem[0])

        def phase(j, rb, si, fire_gather, fire_idx):
            if fire_gather:
                pltpu.make_async_copy(dummy_i, sb[(si + 1) % 4],
                                      ssem[(si + 1) % 4]).wait()
                pltpu.async_copy(x_hbm.at[sb[(si + 1) % 4]], rows[1 - rb],
                                 gsem[1 - rb])
            pltpu.make_async_copy(dummy_r, rows[rb], gsem[rb]).wait()
            if fire_idx:
                pltpu.async_copy(src_hbm.at[pl.ds(ebase + (j + 4) * C, C)],
                                 sb[si], ssem[si])
            pltpu.sync_copy(rows[rb], acc.at[didx_v.at[j]], add=True)

        peel_start = iters - 4 - (iters % 4)

        def step(k, _):
            j = 4 * k
            phase(j, 0, 0, True, True)
            phase(j + 1, 1, 1, True, True)
            phase(j + 2, 0, 2, True, True)
            phase(j + 3, 1, 3, True, True)
            return 0
        lax.fori_loop(0, peel_start // 4, step, 0)

        for j in range(peel_start, iters):
            phase(j, j % 2, j % 4, j + 1 < iters, j + 4 < iters)
        plsc.subcore_barrier()

        pltpu.sync_copy(acc.at[pl.ds(row0, rows_per_sub)],
                        agg_out.at[cid, pl.ds(row0, rows_per_sub)])

    return pl.kernel(
        body,
        out_type=jax.ShapeDtypeStruct((NC, n, d), jnp.float32),
        mesh=mesh,
        scratch_types=[
            pltpu.VMEM((C,), jnp.int32),
            pltpu.VMEM((C,), jnp.int32),
            pltpu.VMEM((C,), jnp.int32),
            pltpu.VMEM((C,), jnp.int32),
            pltpu.VMEM((iters, C), jnp.int32),
            pltpu.VMEM((C, d), jnp.float32),
            pltpu.VMEM((C, d), jnp.float32),
            pltpu.VMEM_SHARED((n, d), jnp.float32),
            pltpu.SemaphoreType.DMA,
            pltpu.SemaphoreType.DMA,
            pltpu.SemaphoreType.DMA,
            pltpu.SemaphoreType.DMA,
            pltpu.SemaphoreType.DMA,
            pltpu.SemaphoreType.DMA,
        ],
    )


def _make_sc_counts(n, iters, d):
    rows_per_sub = n // NS
    nz = rows_per_sub // C
    burst = 5
    nb = iters // burst

    mesh = plsc.VectorSubcoreMesh(core_axis_name="c", subcore_axis_name="s",
                                  num_cores=NC, num_subcores=NS)

    def body(dst_hbm, cnt_out, didx_v, ones_v, cacc, sem_s):
        cid = lax.axis_index("c")
        sid = lax.axis_index("s")
        wid = cid * NS + sid

        pltpu.sync_copy(dst_hbm.at[wid], didx_v)

        _fill(ones_v, C, d, 0.0)
        row0 = sid * rows_per_sub

        def zero_acc(k, _):
            pltpu.sync_copy(ones_v, cacc.at[pl.ds(row0 + k * C, C)])
            return 0
        lax.fori_loop(0, nz, zero_acc, 0)
        _fill(ones_v, C, d, 1.0)
        plsc.subcore_barrier()

        def step(k, _):
            descs = [
                pltpu.async_copy(ones_v, cacc.at[didx_v.at[burst * k + t]],
                                 sem_s, add=True)
                for t in range(burst)
            ]
            for dsc in descs:
                dsc.wait()
            return 0
        lax.fori_loop(0, nb, step, 0)
        plsc.subcore_barrier()

        pltpu.sync_copy(cacc.at[pl.ds(row0, rows_per_sub)],
                        cnt_out.at[cid, pl.ds(row0, rows_per_sub)])

    return pl.kernel(
        body,
        out_type=jax.ShapeDtypeStruct((NC, n, d), jnp.float32),
        mesh=mesh,
        scratch_types=[
            pltpu.VMEM((iters, C), jnp.int32),
            pltpu.VMEM((C, d), jnp.float32),
            pltpu.VMEM_SHARED((n, d), jnp.float32),
            pltpu.SemaphoreType.DMA,
        ],
    )


def _dense(aggp, cntp, xin, wl, wr, b):
    n, d = xin.shape
    h = wl.shape[1]
    blk = min(1024, n)
    grid = (n // blk,)

    def body(aggp_ref, cnt_ref, x_ref, wl_ref, wr_ref, b_ref, o_ref):
        agg = aggp_ref[0] + aggp_ref[1]
        cnt = cnt_ref[0] + cnt_ref[1]
        mean = agg / jnp.maximum(cnt[:, :1], 1.0)
        acc = jnp.dot(mean, wl_ref[...], preferred_element_type=jnp.float32)
        acc = acc + jnp.dot(x_ref[...], wr_ref[...],
                            preferred_element_type=jnp.float32)
        acc = acc + b_ref[...]
        o_ref[...] = jnp.maximum(acc, 0.0)

    return pl.pallas_call(
        body,
        grid=grid,
        in_specs=[
            pl.BlockSpec((NC, blk, h), lambda i: (0, i, 0)),
            pl.BlockSpec((NC, blk, h), lambda i: (0, i, 0)),
            pl.BlockSpec((blk, d), lambda i: (i, 0)),
            pl.BlockSpec((d, h), lambda i: (0, 0)),
            pl.BlockSpec((d, h), lambda i: (0, 0)),
            pl.BlockSpec((1, h), lambda i: (0, 0)),
        ],
        out_specs=pl.BlockSpec((blk, h), lambda i: (i, 0)),
        out_shape=jax.ShapeDtypeStruct((n, h), jnp.float32),
    )(aggp, cntp, xin, wl, wr, b.reshape(1, h))


def kernel(x, edge_index, Wl1, Wr1, b1, Wl2, Wr2, b2):
    n, d = x.shape
    e = edge_index.shape[1]
    nw = NC * NS
    ew = e // nw
    iters = (ew + C - 1) // C
    ewp = iters * C

    npad = ((n + NS * 8 - 1) // (NS * 8)) * (NS * 8)
    npad = max(npad, ((n + 1023) // 1024) * 1024)
    x_p = jnp.pad(x, ((0, npad - n), (0, 0)))

    srcw = jnp.pad(edge_index[0].reshape(nw, ew),
                   ((0, 0), (0, ewp - ew))).reshape(nw * ewp)
    dstw = jnp.pad(edge_index[1].reshape(nw, ew), ((0, 0), (0, ewp - ew)),
                   constant_values=n)
    dst3 = dstw.reshape(nw, iters, C)

    cnt = _make_sc_counts(npad, iters, d)(dst3)
    agg1 = _make_sc_agg(npad, iters, d)(x_p, srcw, dst3)
    h1 = _dense(agg1, cnt, x_p, Wl1, Wr1, b1)
    agg2 = _make_sc_agg(npad, iters, Wl1.shape[1])(h1, srcw, dst3)
    out = _dense(agg2, cnt, h1, Wl2, Wr2, b2)
    return out[:n]


# speedup vs baseline: 2.6253x; 1.0121x over previous
import jax
import jax.numpy as jnp
from jax import lax
from jax.experimental import pallas as pl
from jax.experimental.pallas import tpu as pltpu
from jax.experimental.pallas import tpu_sc as plsc

NC = 2
NS = 16
C = 80


def _fill(ref, rows, d, val):
    v16 = jnp.full((16,), val, jnp.float32)

    def fr(r, _):
        def fc(i, _):
            ref[r, pl.ds(i * 16, 16)] = v16
            return 0
        lax.fori_loop(0, d // 16, fc, 0)
        return 0
    lax.fori_loop(0, rows, fr, 0)


def _make_sc_agg(n, iters, d, with_counts):
    rows_per_sub = n // NS
    nz = rows_per_sub // C
    burst = 5

    mesh = plsc.VectorSubcoreMesh(core_axis_name="c", subcore_axis_name="s",
                                  num_cores=NC, num_subcores=NS)

    def body(x_hbm, src_hbm, dst_hbm, agg_out, *rest):
        if with_counts:
            (cnt_out, sb0, sb1, sb2, sb3, didx_v, rows_0, rows_1, acc,
             ss0, ss1, ss2, ss3, gsem_0, gsem_1, csem) = rest
        else:
            (sb0, sb1, sb2, sb3, didx_v, rows_0, rows_1, acc,
             ss0, ss1, ss2, ss3, gsem_0, gsem_1) = rest
        cid = lax.axis_index("c")
        sid = lax.axis_index("s")
        wid = cid * NS + sid

        sb = (sb0, sb1, sb2, sb3)
        ssem = (ss0, ss1, ss2, ss3)
        rows = (rows_0, rows_1)
        gsem = (gsem_0, gsem_1)

        pltpu.sync_copy(dst_hbm.at[wid], didx_v)

        row0 = sid * rows_per_sub

        def zero_acc_pass():
            _fill(rows_0, C, d, 0.0)

            def zero_acc(k, _):
                pltpu.sync_copy(rows_0, acc.at[pl.ds(row0 + k * C, C)])
                return 0
            lax.fori_loop(0, nz, zero_acc, 0)
            plsc.subcore_barrier()

        zero_acc_pass()

        if with_counts:
            _fill(rows_0, C, d, 1.0)

            def cstep(k, _):
                descs = [
                    pltpu.async_copy(rows_0,
                                     acc.at[didx_v.at[burst * k + t]],
                                     csem, add=True)
                    for t in range(burst)
                ]
                for dsc in descs:
                    dsc.wait()
                return 0
            lax.fori_loop(0, iters // burst, cstep, 0)
            plsc.subcore_barrier()
            pltpu.sync_copy(acc.at[pl.ds(row0, rows_per_sub)],
                            cnt_out.at[cid, pl.ds(row0, rows_per_sub)])
            zero_acc_pass()

        dummy_r = x_hbm.at[pl.ds(0, C)]
        dummy_i = src_hbm.at[pl.ds(0, C)]
        ebase = wid * iters * C

        pltpu.sync_copy(src_hbm.at[pl.ds(ebase, C)], sb[0])
        for t in (1, 2, 3):
            pltpu.async_copy(src_hbm.at[pl.ds(ebase + t * C, C)], sb[t],
                             ssem[t])
        pltpu.async_copy(x_hbm.at[sb[0]], rows_0, gsem[0])

        def phase(j, rb, si, fire_gather, fire_idx):
            if fire_gather:
                pltpu.make_async_copy(dummy_i, sb[(si + 1) % 4],
                                      ssem[(si + 1) % 4]).wait()
                pltpu.async_copy(x_hbm.at[sb[(si + 1) % 4]], rows[1 - rb],
                                 gsem[1 - rb])
            pltpu.make_async_copy(dummy_r, rows[rb], gsem[rb]).wait()
            if fire_idx:
                pltpu.async_copy(src_hbm.at[pl.ds(ebase + (j + 4) * C, C)],
                                 sb[si], ssem[si])
            pltpu.sync_copy(rows[rb], acc.at[didx_v.at[j]], add=True)

        peel_start = iters - 4 - (iters % 4)

        def step(k, _):
            j = 4 * k
            phase(j, 0, 0, True, True)
            phase(j + 1, 1, 1, True, True)
            phase(j + 2, 0, 2, True, True)
            phase(j + 3, 1, 3, True, True)
            return 0
        lax.fori_loop(0, peel_start // 4, step, 0)

        for j in range(peel_start, iters):
            phase(j, j % 2, j % 4, j + 1 < iters, j + 4 < iters)
        plsc.subcore_barrier()

        pltpu.sync_copy(acc.at[pl.ds(row0, rows_per_sub)],
                        agg_out.at[cid, pl.ds(row0, rows_per_sub)])

    out_type = jax.ShapeDtypeStruct((NC, n, d), jnp.float32)
    scratch = [
        pltpu.VMEM((C,), jnp.int32),
        pltpu.VMEM((C,), jnp.int32),
        pltpu.VMEM((C,), jnp.int32),
        pltpu.VMEM((C,), jnp.int32),
        pltpu.VMEM((iters, C), jnp.int32),
        pltpu.VMEM((C, d), jnp.float32),
        pltpu.VMEM((C, d), jnp.float32),
        pltpu.VMEM_SHARED((n, d), jnp.float32),
        pltpu.SemaphoreType.DMA,
        pltpu.SemaphoreType.DMA,
        pltpu.SemaphoreType.DMA,
        pltpu.SemaphoreType.DMA,
        pltpu.SemaphoreType.DMA,
        pltpu.SemaphoreType.DMA,
    ]
    if with_counts:
        out_type = [out_type, jax.ShapeDtypeStruct((NC, n, d), jnp.float32)]
        scratch = scratch + [pltpu.SemaphoreType.DMA]

    return pl.kernel(body, out_type=out_type, mesh=mesh,
                     scratch_types=scratch)


def _dense(aggp, cntp, xin, wl, wr, b):
    n, d = xin.shape
    h = wl.shape[1]
    blk = min(1024, n)
    grid = (n // blk,)

    def body(aggp_ref, cnt_ref, x_ref, wl_ref, wr_ref, b_ref, o_ref):
        agg = aggp_ref[0] + aggp_ref[1]
        cnt = cnt_ref[0] + cnt_ref[1]
        mean = agg / jnp.maximum(cnt[:, :1], 1.0)
        acc = jnp.dot(mean, wl_ref[...], preferred_element_type=jnp.float32)
        acc = acc + jnp.dot(x_ref[...], wr_ref[...],
                            preferred_element_type=jnp.float32)
        acc = acc + b_ref[...]
        o_ref[...] = jnp.maximum(acc, 0.0)

    return pl.pallas_call(
        body,
        grid=grid,
        in_specs=[
            pl.BlockSpec((NC, blk, h), lambda i: (0, i, 0)),
            pl.BlockSpec((NC, blk, h), lambda i: (0, i, 0)),
            pl.BlockSpec((blk, d), lambda i: (i, 0)),
            pl.BlockSpec((d, h), lambda i: (0, 0)),
            pl.BlockSpec((d, h), lambda i: (0, 0)),
            pl.BlockSpec((1, h), lambda i: (0, 0)),
        ],
        out_specs=pl.BlockSpec((blk, h), lambda i: (i, 0)),
        out_shape=jax.ShapeDtypeStruct((n, h), jnp.float32),
    )(aggp, cntp, xin, wl, wr, b.reshape(1, h))


def kernel(x, edge_index, Wl1, Wr1, b1, Wl2, Wr2, b2):
    n, d = x.shape
    e = edge_index.shape[1]
    nw = NC * NS
    ew = e // nw
    iters = (ew + C - 1) // C
    ewp = iters * C

    npad = ((n + NS * 8 - 1) // (NS * 8)) * (NS * 8)
    npad = max(npad, ((n + 1023) // 1024) * 1024)
    x_p = jnp.pad(x, ((0, npad - n), (0, 0)))

    srcw = jnp.pad(edge_index[0].reshape(nw, ew),
                   ((0, 0), (0, ewp - ew))).reshape(nw * ewp)
    dstw = jnp.pad(edge_index[1].reshape(nw, ew), ((0, 0), (0, ewp - ew)),
                   constant_values=n)
    dst3 = dstw.reshape(nw, iters, C)

    agg1, cnt = _make_sc_agg(npad, iters, d, True)(x_p, srcw, dst3)
    h1 = _dense(agg1, cnt, x_p, Wl1, Wr1, b1)
    agg2 = _make_sc_agg(npad, iters, Wl1.shape[1], False)(h1, srcw, dst3)
    out = _dense(agg2, cnt, h1, Wl2, Wr2, b2)
    return out[:n]
